# baseline (speedup 1.0000x reference)
import jax
import jax.numpy as jnp
from jax.experimental import pallas as pl
from jax.experimental.pallas import tpu as pltpu

_T = 2048
_D = 2048
_H = 16
_W = 128
_SCALE = 1.0 / (_W ** 0.5)
_MBLK = 1024
_NCHUNK = 256
_NSTEPS = _D // _NCHUNK
_OCHUNK = 1024
_OSTEPS = _D // _OCHUNK

_DN_T = (((1,), (1,)), ((), ()))


def _fused_kernel(x_ref, wq_ref, wk_ref, wv_ref, wo_ref, o_ref,
                  attn_ref, q_ref, k_ref, v_ref):
    j = pl.program_id(1)

    @pl.when(j <= _NSTEPS)
    def _main():
        qb = q_ref[...]
        kb = k_ref[...]
        vb = v_ref[...]
        rows = jax.lax.broadcasted_iota(jnp.int32, (2 * _W, 2 * _W), 0)
        cols = jax.lax.broadcasted_iota(jnp.int32, (2 * _W, 2 * _W), 1)
        mask = (rows // _W) == (cols // _W)
        for bp in range(_MBLK // (2 * _W)):
            rs = slice(bp * 2 * _W, (bp + 1) * 2 * _W)
            for h in range(_NCHUNK // _W):
                cs = slice(h * _W, (h + 1) * _W)
                qs = qb[rs, cs]
                ks = kb[rs, cs]
                vs = vb[rs, cs]
                s = jax.lax.dot_general(
                    qs, ks, _DN_T, preferred_element_type=jnp.float32)
                e = jnp.where(mask, jnp.exp2(jnp.minimum(s, 86.0)), 0.0)
                ob = jax.lax.dot_general(
                    e.astype(jnp.bfloat16), vs, (((1,), (0,)), ((), ())),
                    preferred_element_type=jnp.float32)
                r = 1.0 / jnp.sum(e, axis=-1, keepdims=True)
                attn_ref[rs, pl.ds(j * _NCHUNK + h * _W, _W)] = (
                    (ob * r).astype(jnp.bfloat16))

        xb = x_ref[...]
        wqc = wq_ref[...].astype(jnp.bfloat16)
        wkc = wk_ref[...].astype(jnp.bfloat16)
        wvc = wv_ref[...].astype(jnp.bfloat16)
        q = jax.lax.dot_general(xb, wqc, _DN_T,
                                preferred_element_type=jnp.float32)
        k = jax.lax.dot_general(xb, wkc, _DN_T,
                                preferred_element_type=jnp.float32)
        v = jax.lax.dot_general(xb, wvc, _DN_T,
                                preferred_element_type=jnp.float32)
        q_ref[...] = (q * (_SCALE * 1.4426950408889634)).astype(jnp.bfloat16)
        k_ref[...] = k.astype(jnp.bfloat16)
        v_ref[...] = v.astype(jnp.bfloat16)

    @pl.when(j > _NSTEPS)
    def _project_out():
        woc = wo_ref[...].astype(jnp.bfloat16)
        o_ref[...] = jax.lax.dot_general(
            attn_ref[:, _NCHUNK:], woc, _DN_T,
            preferred_element_type=jnp.float32)


@jax.jit
def _run(x2d, wq, wk, wv, wo):
    nj = _NSTEPS + 1 + _OSTEPS
    wspec = pl.BlockSpec(
        (_NCHUNK, _D), lambda i, j: (jnp.minimum(j, _NSTEPS - 1), 0))
    return pl.pallas_call(
        _fused_kernel,
        grid=(_T // _MBLK, nj),
        in_specs=[
            pl.BlockSpec((_MBLK, _D), lambda i, j: (i, 0)),
            wspec, wspec, wspec,
            pl.BlockSpec(
                (_OCHUNK, _D),
                lambda i, j: (jnp.clip(j - _NSTEPS - 1, 0, _OSTEPS - 1), 0)),
        ],
        out_specs=pl.BlockSpec(
            (_MBLK, _OCHUNK),
            lambda i, j: (i, jnp.clip(j - _NSTEPS - 1, 0, _OSTEPS - 1))),
        out_shape=jax.ShapeDtypeStruct((_T, _D), jnp.float32),
        scratch_shapes=[
            pltpu.VMEM((_MBLK, _D + _NCHUNK), jnp.bfloat16),
            pltpu.VMEM((_MBLK, _NCHUNK), jnp.bfloat16),
            pltpu.VMEM((_MBLK, _NCHUNK), jnp.bfloat16),
            pltpu.VMEM((_MBLK, _NCHUNK), jnp.bfloat16),
        ],
        compiler_params=pltpu.CompilerParams(
            dimension_semantics=("parallel", "arbitrary")),
    )(x2d, wq, wk, wv, wo)


def kernel(x, Wq, Wk, Wv, Wo):
    B = x.shape[0]
    x2d = x.reshape(_T, _D).astype(jnp.bfloat16)
    return _run(x2d, Wq, Wk, Wv, Wo).reshape(B, _T, _D)


# speedup vs baseline: 1.0103x; 1.0103x over previous
import jax
import jax.numpy as jnp
from jax.experimental import pallas as pl
from jax.experimental.pallas import tpu as pltpu

_T = 2048
_D = 2048
_H = 16
_W = 128
_SCALE = 1.0 / (_W ** 0.5)
_MBLK = 1024
_NCHUNK = 256
_NSTEPS = _D // _NCHUNK
_OCHUNK = 1024
_OSTEPS = _D // _OCHUNK

_DN_T = (((1,), (1,)), ((), ()))


def _fused_kernel(x_ref, wq_ref, wk_ref, wv_ref, wo_ref, o_ref,
                  attn_ref, q_ref, k_ref, v_ref):
    j = pl.program_id(1)

    @pl.when(j <= _NSTEPS)
    def _main():
        qb = q_ref[...]
        kb = k_ref[...]
        vb = v_ref[...]
        rows = jax.lax.broadcasted_iota(jnp.int32, (2 * _W, 2 * _W), 0)
        cols = jax.lax.broadcasted_iota(jnp.int32, (2 * _W, 2 * _W), 1)
        mask = (rows // _W) == (cols // _W)
        pend = []
        for bp in range(_MBLK // (2 * _W)):
            rs = slice(bp * 2 * _W, (bp + 1) * 2 * _W)
            for h in range(_NCHUNK // _W):
                cs = slice(h * _W, (h + 1) * _W)
                s = jax.lax.dot_general(
                    qb[rs, cs], kb[rs, cs], _DN_T,
                    preferred_element_type=jnp.float32)
                e = jnp.where(mask, jnp.exp2(jnp.minimum(s, 86.0)), 0.0)
                pend.append((rs, cs, h, e))
        for rs, cs, h, e in pend:
            ob = jax.lax.dot_general(
                e.astype(jnp.bfloat16), vb[rs, cs], (((1,), (0,)), ((), ())),
                preferred_element_type=jnp.float32)
            r = 1.0 / jnp.sum(e, axis=-1, keepdims=True)
            attn_ref[rs, pl.ds(j * _NCHUNK + h * _W, _W)] = (
                (ob * r).astype(jnp.bfloat16))

        xb = x_ref[...]
        wqc = wq_ref[...].astype(jnp.bfloat16)
        wkc = wk_ref[...].astype(jnp.bfloat16)
        wvc = wv_ref[...].astype(jnp.bfloat16)
        q = jax.lax.dot_general(xb, wqc, _DN_T,
                                preferred_element_type=jnp.float32)
        k = jax.lax.dot_general(xb, wkc, _DN_T,
                                preferred_element_type=jnp.float32)
        v = jax.lax.dot_general(xb, wvc, _DN_T,
                                preferred_element_type=jnp.float32)
        q_ref[...] = (q * (_SCALE * 1.4426950408889634)).astype(jnp.bfloat16)
        k_ref[...] = k.astype(jnp.bfloat16)
        v_ref[...] = v.astype(jnp.bfloat16)

    @pl.when(j > _NSTEPS)
    def _project_out():
        woc = wo_ref[...].astype(jnp.bfloat16)
        o_ref[...] = jax.lax.dot_general(
            attn_ref[:, _NCHUNK:], woc, _DN_T,
            preferred_element_type=jnp.float32)


@jax.jit
def _run(x2d, wq, wk, wv, wo):
    nj = _NSTEPS + 1 + _OSTEPS
    wspec = pl.BlockSpec(
        (_NCHUNK, _D), lambda i, j: (jnp.minimum(j, _NSTEPS - 1), 0))
    return pl.pallas_call(
        _fused_kernel,
        grid=(_T // _MBLK, nj),
        in_specs=[
            pl.BlockSpec((_MBLK, _D), lambda i, j: (i, 0)),
            wspec, wspec, wspec,
            pl.BlockSpec(
                (_OCHUNK, _D),
                lambda i, j: (jnp.clip(j - _NSTEPS - 1, 0, _OSTEPS - 1), 0)),
        ],
        out_specs=pl.BlockSpec(
            (_MBLK, _OCHUNK),
            lambda i, j: (i, jnp.clip(j - _NSTEPS - 1, 0, _OSTEPS - 1))),
        out_shape=jax.ShapeDtypeStruct((_T, _D), jnp.float32),
        scratch_shapes=[
            pltpu.VMEM((_MBLK, _D + _NCHUNK), jnp.bfloat16),
            pltpu.VMEM((_MBLK, _NCHUNK), jnp.bfloat16),
            pltpu.VMEM((_MBLK, _NCHUNK), jnp.bfloat16),
            pltpu.VMEM((_MBLK, _NCHUNK), jnp.bfloat16),
        ],
        compiler_params=pltpu.CompilerParams(
            dimension_semantics=("parallel", "arbitrary")),
    )(x2d, wq, wk, wv, wo)


def kernel(x, Wq, Wk, Wv, Wo):
    B = x.shape[0]
    x2d = x.reshape(_T, _D).astype(jnp.bfloat16)
    return _run(x2d, Wq, Wk, Wv, Wo).reshape(B, _T, _D)


# speedup vs baseline: 1.0427x; 1.0321x over previous
import jax
import jax.numpy as jnp
from jax.experimental import pallas as pl
from jax.experimental.pallas import tpu as pltpu

_T = 2048
_D = 2048
_H = 16
_W = 128
_SCALE = 1.0 / (_W ** 0.5)
_MBLK = 1024
_NCHUNK = 256
_NSTEPS = _D // _NCHUNK
_OCHUNK = 1024
_OSTEPS = _D // _OCHUNK

_DN_T = (((1,), (1,)), ((), ()))


def _fused_kernel(x_ref, wq_ref, wk_ref, wv_ref, wo_ref, o_ref,
                  attn_ref, q_ref, k_ref, v_ref):
    j = pl.program_id(1)

    @pl.when((j >= 1) & (j <= _NSTEPS))
    def _attend():
        qb = q_ref[...]
        kb = k_ref[...]
        vb = v_ref[...]
        rows = jax.lax.broadcasted_iota(jnp.int32, (2 * _W, 2 * _W), 0)
        cols = jax.lax.broadcasted_iota(jnp.int32, (2 * _W, 2 * _W), 1)
        mask = (rows // _W) == (cols // _W)
        pend = []
        for bp in range(_MBLK // (2 * _W)):
            rs = slice(bp * 2 * _W, (bp + 1) * 2 * _W)
            for h in range(_NCHUNK // _W):
                cs = slice(h * _W, (h + 1) * _W)
                s = jax.lax.dot_general(
                    qb[rs, cs], kb[rs, cs], _DN_T,
                    preferred_element_type=jnp.float32)
                e = jnp.where(mask, jnp.exp2(jnp.minimum(s, 86.0)), 0.0)
                pend.append((rs, cs, h, e))
        for rs, cs, h, e in pend:
            ob = jax.lax.dot_general(
                e.astype(jnp.bfloat16), vb[rs, cs], (((1,), (0,)), ((), ())),
                preferred_element_type=jnp.float32)
            r = 1.0 / jnp.sum(e, axis=-1, keepdims=True)
            attn_ref[rs, pl.ds(j * _NCHUNK + h * _W, _W)] = (
                (ob * r).astype(jnp.bfloat16))

    @pl.when(j < _NSTEPS)
    def _project_qkv():
        xb = x_ref[...]
        wqc = wq_ref[...].astype(jnp.bfloat16)
        wkc = wk_ref[...].astype(jnp.bfloat16)
        wvc = wv_ref[...].astype(jnp.bfloat16)
        q = jax.lax.dot_general(xb, wqc, _DN_T,
                                preferred_element_type=jnp.float32)
        k = jax.lax.dot_general(xb, wkc, _DN_T,
                                preferred_element_type=jnp.float32)
        v = jax.lax.dot_general(xb, wvc, _DN_T,
                                preferred_element_type=jnp.float32)
        q_ref[...] = (q * (_SCALE * 1.4426950408889634)).astype(jnp.bfloat16)
        k_ref[...] = k.astype(jnp.bfloat16)
        v_ref[...] = v.astype(jnp.bfloat16)

    @pl.when(j > _NSTEPS)
    def _project_out():
        woc = wo_ref[...].astype(jnp.bfloat16)
        o_ref[...] = jax.lax.dot_general(
            attn_ref[:, _NCHUNK:], woc, _DN_T,
            preferred_element_type=jnp.float32)


@jax.jit
def _run(x2d, wq, wk, wv, wo):
    nj = _NSTEPS + 1 + _OSTEPS
    wspec = pl.BlockSpec(
        (_NCHUNK, _D), lambda i, j: (jnp.minimum(j, _NSTEPS - 1), 0))
    return pl.pallas_call(
        _fused_kernel,
        grid=(_T // _MBLK, nj),
        in_specs=[
            pl.BlockSpec((_MBLK, _D), lambda i, j: (i, 0)),
            wspec, wspec, wspec,
            pl.BlockSpec(
                (_OCHUNK, _D),
                lambda i, j: (jnp.clip(j - _NSTEPS - 1, 0, _OSTEPS - 1), 0)),
        ],
        out_specs=pl.BlockSpec(
            (_MBLK, _OCHUNK),
            lambda i, j: (i, jnp.clip(j - _NSTEPS - 1, 0, _OSTEPS - 1))),
        out_shape=jax.ShapeDtypeStruct((_T, _D), jnp.float32),
        scratch_shapes=[
            pltpu.VMEM((_MBLK, _D + _NCHUNK), jnp.bfloat16),
            pltpu.VMEM((_MBLK, _NCHUNK), jnp.bfloat16),
            pltpu.VMEM((_MBLK, _NCHUNK), jnp.bfloat16),
            pltpu.VMEM((_MBLK, _NCHUNK), jnp.bfloat16),
        ],
        compiler_params=pltpu.CompilerParams(
            dimension_semantics=("parallel", "arbitrary")),
    )(x2d, wq, wk, wv, wo)


def kernel(x, Wq, Wk, Wv, Wo):
    B = x.shape[0]
    x2d = x.reshape(_T, _D).astype(jnp.bfloat16)
    return _run(x2d, Wq, Wk, Wv, Wo).reshape(B, _T, _D)


# speedup vs baseline: 1.1119x; 1.0664x over previous
import jax
import jax.numpy as jnp
from jax.experimental import pallas as pl
from jax.experimental.pallas import tpu as pltpu

_T = 2048
_D = 2048
_H = 16
_W = 128
_SCALE = 1.0 / (_W ** 0.5)
_NCHUNK = 256
_NSTEPS = _D // _NCHUNK
_OCHUNK = 256
_OSTEPS = _D // _OCHUNK
_WAVE = 8

_DN_T = (((1,), (1,)), ((), ()))


def _fused_kernel(x_ref, wq_ref, wk_ref, wv_ref, wo_ref, o_ref,
                  attn_ref, q_ref, k_ref, v_ref):
    j = pl.program_id(0)

    @pl.when((j >= 1) & (j <= _NSTEPS))
    def _attend():
        qb = q_ref[...]
        kb = k_ref[...]
        vb = v_ref[...]
        rows = jax.lax.broadcasted_iota(jnp.int32, (2 * _W, 2 * _W), 0)
        cols = jax.lax.broadcasted_iota(jnp.int32, (2 * _W, 2 * _W), 1)
        mask = (rows // _W) == (cols // _W)

        def emit_ob(wave):
            for rs, h, e in wave:
                ob = jax.lax.dot_general(
                    e.astype(jnp.bfloat16), vb[rs, h * _W:(h + 1) * _W],
                    (((1,), (0,)), ((), ())),
                    preferred_element_type=jnp.float32)
                r = 1.0 / jnp.sum(e, axis=-1, keepdims=True)
                attn_ref[rs, pl.ds((j - 1) * _NCHUNK + h * _W, _W)] = (
                    (ob * r).astype(jnp.bfloat16))

        iters = [(slice(bp * 2 * _W, (bp + 1) * 2 * _W), h)
                 for bp in range(_T // (2 * _W))
                 for h in range(_NCHUNK // _W)]
        prev = None
        for w0 in range(0, len(iters), _WAVE):
            cur = []
            for rs, h in iters[w0:w0 + _WAVE]:
                cs = slice(h * _W, (h + 1) * _W)
                s = jax.lax.dot_general(
                    qb[rs, cs], kb[rs, cs], _DN_T,
                    preferred_element_type=jnp.float32)
                e = jnp.where(mask, jnp.exp2(jnp.minimum(s, 86.0)), 0.0)
                cur.append((rs, h, e))
            if prev is not None:
                emit_ob(prev)
            prev = cur
        emit_ob(prev)

    @pl.when(j < _NSTEPS)
    def _project_qkv():
        xb = x_ref[...]
        wqc = wq_ref[...].astype(jnp.bfloat16)
        wkc = wk_ref[...].astype(jnp.bfloat16)
        wvc = wv_ref[...].astype(jnp.bfloat16)
        q = jax.lax.dot_general(xb, wqc, _DN_T,
                                preferred_element_type=jnp.float32)
        k = jax.lax.dot_general(xb, wkc, _DN_T,
                                preferred_element_type=jnp.float32)
        v = jax.lax.dot_general(xb, wvc, _DN_T,
                                preferred_element_type=jnp.float32)
        q_ref[...] = (q * (_SCALE * 1.4426950408889634)).astype(jnp.bfloat16)
        k_ref[...] = k.astype(jnp.bfloat16)
        v_ref[...] = v.astype(jnp.bfloat16)

    @pl.when(j > _NSTEPS)
    def _project_out():
        woc = wo_ref[...].astype(jnp.bfloat16)
        o_ref[...] = jax.lax.dot_general(
            attn_ref[...], woc, _DN_T, preferred_element_type=jnp.float32)


@jax.jit
def _run(x2d, wq, wk, wv, wo):
    nj = _NSTEPS + 1 + _OSTEPS
    wspec = pl.BlockSpec(
        (_NCHUNK, _D), lambda j: (jnp.minimum(j, _NSTEPS - 1), 0))
    return pl.pallas_call(
        _fused_kernel,
        grid=(nj,),
        in_specs=[
            pl.BlockSpec((_T, _D), lambda j: (0, 0)),
            wspec, wspec, wspec,
            pl.BlockSpec(
                (_OCHUNK, _D),
                lambda j: (jnp.clip(j - _NSTEPS - 1, 0, _OSTEPS - 1), 0)),
        ],
        out_specs=pl.BlockSpec(
            (_T, _OCHUNK),
            lambda j: (0, jnp.clip(j - _NSTEPS - 1, 0, _OSTEPS - 1))),
        out_shape=jax.ShapeDtypeStruct((_T, _D), jnp.float32),
        scratch_shapes=[
            pltpu.VMEM((_T, _D), jnp.bfloat16),
            pltpu.VMEM((_T, _NCHUNK), jnp.bfloat16),
            pltpu.VMEM((_T, _NCHUNK), jnp.bfloat16),
            pltpu.VMEM((_T, _NCHUNK), jnp.bfloat16),
        ],
        compiler_params=pltpu.CompilerParams(
            dimension_semantics=("arbitrary",)),
    )(x2d, wq, wk, wv, wo)


def kernel(x, Wq, Wk, Wv, Wo):
    B = x.shape[0]
    x2d = x.reshape(_T, _D).astype(jnp.bfloat16)
    return _run(x2d, Wq, Wk, Wv, Wo).reshape(B, _T, _D)


# speedup vs baseline: 1.2005x; 1.0796x over previous
import jax
import jax.numpy as jnp
from jax.experimental import pallas as pl
from jax.experimental.pallas import tpu as pltpu

_T = 2048
_D = 2048
_H = 16
_W = 128
_SCALE = 1.0 / (_W ** 0.5)
_NCHUNK = 256
_NSTEPS = _D // _NCHUNK
_OCHUNK = 256
_OSTEPS = _D // _OCHUNK
_WAVE = 8

_DN_T = (((1,), (1,)), ((), ()))


def _fused_kernel(x_ref, wq_ref, wk_ref, wv_ref, wo_ref, o_ref,
                  attn_ref, q_ref, k_ref, v_ref):
    j = pl.program_id(0)

    @pl.when((j >= 1) & (j <= _NSTEPS))
    def _attend():
        qb = q_ref[...]
        kb = k_ref[...]
        vb = v_ref[...]
        rows = jax.lax.broadcasted_iota(jnp.int32, (2 * _W, 2 * _W), 0)
        cols = jax.lax.broadcasted_iota(jnp.int32, (2 * _W, 2 * _W), 1)
        mask = (rows // _W) == (cols // _W)

        def emit_ob(wave):
            for rs, h, e in wave:
                ob = jax.lax.dot_general(
                    e.astype(jnp.bfloat16), vb[rs, h * _W:(h + 1) * _W],
                    (((1,), (0,)), ((), ())),
                    preferred_element_type=jnp.float32)
                r = 1.0 / jnp.sum(e, axis=-1, keepdims=True)
                attn_ref[rs, pl.ds((j - 1) * _NCHUNK + h * _W, _W)] = (
                    (ob * r).astype(jnp.bfloat16))

        iters = [(slice(bp * 2 * _W, (bp + 1) * 2 * _W), h)
                 for bp in range(_T // (2 * _W))
                 for h in range(_NCHUNK // _W)]
        prev = None
        for w0 in range(0, len(iters), _WAVE):
            cur = []
            for rs, h in iters[w0:w0 + _WAVE]:
                cs = slice(h * _W, (h + 1) * _W)
                s = jax.lax.dot_general(
                    qb[rs, cs], kb[rs, cs], _DN_T,
                    preferred_element_type=jnp.float32)
                e = jnp.where(mask, jnp.exp2(jnp.minimum(s, 86.0)), 0.0)
                cur.append((rs, h, e))
            if prev is not None:
                emit_ob(prev)
            prev = cur
        emit_ob(prev)

    @pl.when(j < _NSTEPS)
    def _project_qkv():
        wqc = wq_ref[...].astype(jnp.bfloat16)
        wkc = wk_ref[...].astype(jnp.bfloat16)
        wvc = wv_ref[...].astype(jnp.bfloat16)
        for ms in range(0, _T, _T // 2):
            sl = slice(ms, ms + _T // 2)
            xb = x_ref[sl, :].astype(jnp.bfloat16)
            q = jax.lax.dot_general(xb, wqc, _DN_T,
                                    preferred_element_type=jnp.float32)
            k = jax.lax.dot_general(xb, wkc, _DN_T,
                                    preferred_element_type=jnp.float32)
            v = jax.lax.dot_general(xb, wvc, _DN_T,
                                    preferred_element_type=jnp.float32)
            q_ref[sl, :] = (
                q * (_SCALE * 1.4426950408889634)).astype(jnp.bfloat16)
            k_ref[sl, :] = k.astype(jnp.bfloat16)
            v_ref[sl, :] = v.astype(jnp.bfloat16)

    @pl.when(j > _NSTEPS)
    def _project_out():
        woc = wo_ref[...].astype(jnp.bfloat16)
        o_ref[...] = jax.lax.dot_general(
            attn_ref[...], woc, _DN_T, preferred_element_type=jnp.float32)


@jax.jit
def _run(x2d, wq, wk, wv, wo):
    nj = _NSTEPS + 1 + _OSTEPS
    wspec = pl.BlockSpec(
        (_NCHUNK, _D), lambda j: (jnp.minimum(j, _NSTEPS - 1), 0))
    return pl.pallas_call(
        _fused_kernel,
        grid=(nj,),
        in_specs=[
            pl.BlockSpec((_T, _D), lambda j: (0, 0)),
            wspec, wspec, wspec,
            pl.BlockSpec(
                (_OCHUNK, _D),
                lambda j: (jnp.clip(j - _NSTEPS - 1, 0, _OSTEPS - 1), 0)),
        ],
        out_specs=pl.BlockSpec(
            (_T, _OCHUNK),
            lambda j: (0, jnp.clip(j - _NSTEPS - 1, 0, _OSTEPS - 1))),
        out_shape=jax.ShapeDtypeStruct((_T, _D), jnp.float32),
        scratch_shapes=[
            pltpu.VMEM((_T, _D), jnp.bfloat16),
            pltpu.VMEM((_T, _NCHUNK), jnp.bfloat16),
            pltpu.VMEM((_T, _NCHUNK), jnp.bfloat16),
            pltpu.VMEM((_T, _NCHUNK), jnp.bfloat16),
        ],
        compiler_params=pltpu.CompilerParams(
            dimension_semantics=("arbitrary",)),
    )(x2d, wq, wk, wv, wo)


def kernel(x, Wq, Wk, Wv, Wo):
    B = x.shape[0]
    return _run(x.reshape(_T, _D), Wq, Wk, Wv, Wo).reshape(B, _T, _D)


# speedup vs baseline: 1.2215x; 1.0175x over previous
import jax
import jax.numpy as jnp
from jax.experimental import pallas as pl
from jax.experimental.pallas import tpu as pltpu

_T = 2048
_D = 2048
_H = 16
_W = 128
_SCALE = 1.0 / (_W ** 0.5)
_NCHUNK = 256
_NSTEPS = _D // _NCHUNK
_OCHUNK = 256
_OSTEPS = _D // _OCHUNK
_WAVE = 8

_DN_T = (((1,), (1,)), ((), ()))


def _fused_kernel(x_ref, wq_ref, wk_ref, wv_ref, wo_ref, o_ref,
                  attn_ref, q_ref, k_ref, v_ref):
    j = pl.program_id(0)

    def _attend():
        qb = q_ref[...]
        kb = k_ref[...]
        vb = v_ref[...]
        rows = jax.lax.broadcasted_iota(jnp.int32, (2 * _W, 2 * _W), 0)
        cols = jax.lax.broadcasted_iota(jnp.int32, (2 * _W, 2 * _W), 1)
        mask = (rows // _W) == (cols // _W)

        def emit_ob(wave):
            for rs, h, e in wave:
                ob = jax.lax.dot_general(
                    e.astype(jnp.bfloat16), vb[rs, h * _W:(h + 1) * _W],
                    (((1,), (0,)), ((), ())),
                    preferred_element_type=jnp.float32)
                r = 1.0 / jnp.sum(e, axis=-1, keepdims=True)
                attn_ref[rs, pl.ds((j - 1) * _NCHUNK + h * _W, _W)] = (
                    (ob * r).astype(jnp.bfloat16))

        iters = [(slice(bp * 2 * _W, (bp + 1) * 2 * _W), h)
                 for bp in range(_T // (2 * _W))
                 for h in range(_NCHUNK // _W)]
        prev = None
        for w0 in range(0, len(iters), _WAVE):
            cur = []
            for rs, h in iters[w0:w0 + _WAVE]:
                cs = slice(h * _W, (h + 1) * _W)
                s = jax.lax.dot_general(
                    qb[rs, cs], kb[rs, cs], _DN_T,
                    preferred_element_type=jnp.float32)
                e = jnp.where(mask, jnp.exp2(jnp.minimum(s, 86.0)), 0.0)
                cur.append((rs, h, e))
            if prev is not None:
                emit_ob(prev)
            prev = cur
        emit_ob(prev)

    def _project_qkv():
        wqc = wq_ref[...].astype(jnp.bfloat16)
        wkc = wk_ref[...].astype(jnp.bfloat16)
        wvc = wv_ref[...].astype(jnp.bfloat16)
        for ms in range(0, _T, _T // 2):
            sl = slice(ms, ms + _T // 2)
            xb = x_ref[sl, :].astype(jnp.bfloat16)
            q = jax.lax.dot_general(xb, wqc, _DN_T,
                                    preferred_element_type=jnp.float32)
            k = jax.lax.dot_general(xb, wkc, _DN_T,
                                    preferred_element_type=jnp.float32)
            v = jax.lax.dot_general(xb, wvc, _DN_T,
                                    preferred_element_type=jnp.float32)
            q_ref[sl, :] = (
                q * (_SCALE * 1.4426950408889634)).astype(jnp.bfloat16)
            k_ref[sl, :] = k.astype(jnp.bfloat16)
            v_ref[sl, :] = v.astype(jnp.bfloat16)

    @pl.when(j == 0)
    def _first():
        _project_qkv()

    @pl.when((j >= 1) & (j < _NSTEPS))
    def _steady():
        _attend()
        _project_qkv()

    @pl.when(j == _NSTEPS)
    def _last_attend():
        _attend()

    @pl.when(j > _NSTEPS)
    def _project_out():
        woc = wo_ref[...].astype(jnp.bfloat16)
        o_ref[...] = jax.lax.dot_general(
            attn_ref[...], woc, _DN_T, preferred_element_type=jnp.float32)


@jax.jit
def _run(x2d, wq, wk, wv, wo):
    nj = _NSTEPS + 1 + _OSTEPS
    wspec = pl.BlockSpec(
        (_NCHUNK, _D), lambda j: (jnp.minimum(j, _NSTEPS - 1), 0))
    return pl.pallas_call(
        _fused_kernel,
        grid=(nj,),
        in_specs=[
            pl.BlockSpec((_T, _D), lambda j: (0, 0)),
            wspec, wspec, wspec,
            pl.BlockSpec(
                (_OCHUNK, _D),
                lambda j: (jnp.clip(j - _NSTEPS - 1, 0, _OSTEPS - 1), 0)),
        ],
        out_specs=pl.BlockSpec(
            (_T, _OCHUNK),
            lambda j: (0, jnp.clip(j - _NSTEPS - 1, 0, _OSTEPS - 1))),
        out_shape=jax.ShapeDtypeStruct((_T, _D), jnp.float32),
        scratch_shapes=[
            pltpu.VMEM((_T, _D), jnp.bfloat16),
            pltpu.VMEM((_T, _NCHUNK), jnp.bfloat16),
            pltpu.VMEM((_T, _NCHUNK), jnp.bfloat16),
            pltpu.VMEM((_T, _NCHUNK), jnp.bfloat16),
        ],
        compiler_params=pltpu.CompilerParams(
            dimension_semantics=("arbitrary",)),
    )(x2d, wq, wk, wv, wo)


def kernel(x, Wq, Wk, Wv, Wo):
    B = x.shape[0]
    return _run(x.reshape(_T, _D), Wq, Wk, Wv, Wo).reshape(B, _T, _D)
